# Initial kernel scaffold; baseline (speedup 1.0000x reference)
import jax
import jax.numpy as jnp
from jax import lax
from jax.experimental import pallas as pl
from jax.experimental.pallas import tpu as pltpu
from jax.experimental.pallas import tpu_sc as plsc

VOCAB = 1000000
EMB = 64
BATCH = 16384
NEG = 20

NC = 2
NS = 16
NW = NC * NS
B_PER_W = BATCH // NW
CB = 64
N_CHUNKS = B_PER_W // CB
NQ = EMB // 16


def _sc_body(word_idx_hbm, dec_idx_hbm, enc_hbm, dec_hbm,
             pos_hbm, neg_hbm,
             widx_v, didx_v, wrows_v, drows_v, pos_v, neg_v, sem):
    wid = lax.axis_index("s") * NC + lax.axis_index("c")

    def chunk_body(c, _):
        base = wid * B_PER_W + c * CB
        pltpu.sync_copy(word_idx_hbm.at[pl.ds(base, CB)], widx_v)
        pltpu.sync_copy(dec_idx_hbm.at[:, pl.ds(base, CB)], didx_v)
        copies = [pltpu.make_async_copy(enc_hbm.at[widx_v], wrows_v, sem)]
        for j in range(1 + NEG):
            copies.append(
                pltpu.make_async_copy(dec_hbm.at[didx_v.at[j]],
                                      drows_v.at[j], sem))
        for cp in copies:
            cp.start()
        for cp in copies:
            cp.wait()

        def elem_body(e, _):
            w = [wrows_v[e, pl.ds(q * 16, 16)] for q in range(NQ)]
            for j in range(1 + NEG):
                r0 = drows_v[j, e, pl.ds(0, 16)] * w[0]
                r1 = drows_v[j, e, pl.ds(16, 16)] * w[1]
                r2 = drows_v[j, e, pl.ds(32, 16)] * w[2]
                r3 = drows_v[j, e, pl.ds(48, 16)] * w[3]
                s = jnp.sum((r0 + r1) + (r2 + r3))
                if j == 0:
                    pos_v[e] = s
                else:
                    neg_v[e, j - 1] = s
            return ()

        lax.fori_loop(0, CB, elem_body, (), unroll=False)

        pltpu.sync_copy(pos_v, pos_hbm.at[pl.ds(base, CB)])
        pltpu.sync_copy(neg_v, neg_hbm.at[pl.ds(base, CB), :])
        return ()

    lax.fori_loop(0, N_CHUNKS, chunk_body, (), unroll=False)


@jax.jit
def _run(word_idx, dec_idx_t, enc, dec):
    mesh = plsc.VectorSubcoreMesh(
        core_axis_name="c", subcore_axis_name="s",
        num_cores=NC, num_subcores=NS)
    f = pl.kernel(
        _sc_body,
        out_type=(
            jax.ShapeDtypeStruct((BATCH,), jnp.float32),
            jax.ShapeDtypeStruct((BATCH, NEG), jnp.float32),
        ),
        mesh=mesh,
        scratch_types=[
            pltpu.VMEM((CB,), jnp.int32),
            pltpu.VMEM((1 + NEG, CB), jnp.int32),
            pltpu.VMEM((CB, EMB), jnp.float32),
            pltpu.VMEM((1 + NEG, CB, EMB), jnp.float32),
            pltpu.VMEM((CB,), jnp.float32),
            pltpu.VMEM((CB, NEG), jnp.float32),
            pltpu.SemaphoreType.DMA,
        ],
    )
    return f(word_idx, dec_idx_t, enc, dec)


def kernel(input_tokens, ctx_tokens, neg_tokens, encoder_weight, decoder_weight):
    word_idx = input_tokens.reshape(BATCH).astype(jnp.int32)
    ctx_idx = ctx_tokens.reshape(1, BATCH).astype(jnp.int32)
    neg_idx_t = neg_tokens.astype(jnp.int32).T
    dec_idx_t = jnp.concatenate([ctx_idx, neg_idx_t], axis=0)
    pos, neg = _run(word_idx, dec_idx_t, encoder_weight, decoder_weight)
    return (pos.reshape(BATCH, 1, 1), neg.reshape(BATCH, 1, NEG))


# speedup vs baseline: 5.2337x; 5.2337x over previous
import jax
import jax.numpy as jnp
from jax import lax
from jax.experimental import pallas as pl
from jax.experimental.pallas import tpu as pltpu
from jax.experimental.pallas import tpu_sc as plsc

VOCAB = 1000000
EMB = 64
BATCH = 16384
NEG = 20

NC = 2
NS = 16
NW = NC * NS
B_PER_W = BATCH // NW
CB = 64
N_CHUNKS = B_PER_W // CB
NQ = EMB // 16


NJ = 1 + NEG


def _sc_body(word_idx_hbm, dec_idx_hbm, enc_hbm, dec_hbm, out_hbm,
             widx_v, didx_v, wrows_v, drows_v, out_v, sem):
    wid = lax.axis_index("s") * NC + lax.axis_index("c")
    lane = lax.iota(jnp.int32, 16)

    def chunk_body(c, _):
        base = wid * B_PER_W + c * CB
        blk = wid * N_CHUNKS + c
        pltpu.sync_copy(word_idx_hbm.at[pl.ds(base, CB)], widx_v)
        pltpu.sync_copy(dec_idx_hbm.at[blk], didx_v)
        copies = [pltpu.make_async_copy(enc_hbm.at[widx_v], wrows_v, sem)]
        for j in range(NJ):
            copies.append(
                pltpu.make_async_copy(dec_hbm.at[didx_v.at[j]],
                                      drows_v.at[j], sem))
        for cp in copies:
            cp.start()
        for cp in copies:
            cp.wait()

        def elem_body(e, _):
            w = [wrows_v[e, pl.ds(q * 16, 16)] for q in range(NQ)]
            acc_lo = jnp.zeros((16,), jnp.float32)
            acc_hi = jnp.zeros((16,), jnp.float32)
            for j in range(NJ):
                r0 = drows_v[j, e, pl.ds(0, 16)] * w[0]
                r1 = drows_v[j, e, pl.ds(16, 16)] * w[1]
                r2 = drows_v[j, e, pl.ds(32, 16)] * w[2]
                r3 = drows_v[j, e, pl.ds(48, 16)] * w[3]
                s = jnp.sum((r0 + r1) + (r2 + r3))
                if j < 16:
                    acc_lo = jnp.where(lane == j, s, acc_lo)
                else:
                    acc_hi = jnp.where(lane == j - 16, s, acc_hi)
            pos0 = e * NJ
            plsc.store_scatter(out_v, [pos0 + lane], acc_lo)
            plsc.store_scatter(out_v, [pos0 + 16 + lane], acc_hi,
                               mask=lane < NJ - 16)
            return ()

        lax.fori_loop(0, CB, elem_body, (), unroll=False)

        pltpu.sync_copy(out_v, out_hbm.at[pl.ds(base * NJ, CB * NJ)])
        return ()

    lax.fori_loop(0, N_CHUNKS, chunk_body, (), unroll=False)


@jax.jit
def _run(word_idx, dec_idx_t, enc, dec):
    mesh = plsc.VectorSubcoreMesh(
        core_axis_name="c", subcore_axis_name="s",
        num_cores=NC, num_subcores=NS)
    f = pl.kernel(
        _sc_body,
        out_type=jax.ShapeDtypeStruct((BATCH * NJ,), jnp.float32),
        mesh=mesh,
        scratch_types=[
            pltpu.VMEM((CB,), jnp.int32),
            pltpu.VMEM((NJ, CB), jnp.int32),
            pltpu.VMEM((CB, EMB), jnp.float32),
            pltpu.VMEM((NJ, CB, EMB), jnp.float32),
            pltpu.VMEM((CB * NJ,), jnp.float32),
            pltpu.SemaphoreType.DMA,
        ],
        compiler_params=pltpu.CompilerParams(
            needs_layout_passes=False, use_tc_tiling_on_sc=False),
    )
    return f(word_idx, dec_idx_t, enc, dec)


def kernel(input_tokens, ctx_tokens, neg_tokens, encoder_weight, decoder_weight):
    word_idx = input_tokens.reshape(BATCH).astype(jnp.int32)
    ctx_idx = ctx_tokens.reshape(1, BATCH).astype(jnp.int32)
    neg_idx_t = neg_tokens.astype(jnp.int32).T
    dec_idx_t = jnp.concatenate([ctx_idx, neg_idx_t], axis=0)
    dec_idx_blk = dec_idx_t.reshape(NJ, BATCH // CB, CB).transpose(1, 0, 2)
    out = _run(word_idx, dec_idx_blk, encoder_weight, decoder_weight)
    out = out.reshape(BATCH, NJ)
    pos = out[:, 0].reshape(BATCH, 1, 1)
    neg = out[:, 1:].reshape(BATCH, 1, NEG)
    return (pos, neg)


# speedup vs baseline: 5.6792x; 1.0851x over previous
import math

import jax
import jax.numpy as jnp
from jax import lax
from jax.experimental import pallas as pl
from jax.experimental.pallas import tpu as pltpu
from jax.experimental.pallas import tpu_sc as plsc

VOCAB = 1000000
EMB = 64
BATCH = 16384
NEG = 20
NJ = 1 + NEG
NR = 1 + NJ

W = 2048
H = W // 2
G = math.ceil(VOCAB / W)

NC = 2
NS = 16
NW = NC * NS
CB = 16
T_CHUNKS = BATCH // CB
CPW = T_CHUNKS // NW
IDX_STRIDE = 1024
OUT_STRIDE = 1024
NPK = NR * CB
OFF0 = NPK
NQ = EMB // 16


def _tc_pack_body(x_ref, o_ref):
    a = x_ref[:, :H].T
    b = x_ref[:, H:].T
    o_ref[...] = jnp.concatenate([a, b], axis=1)


def _tc_pack(xt):
    return pl.pallas_call(
        _tc_pack_body,
        grid=(G,),
        in_specs=[pl.BlockSpec((64, W), lambda g: (0, g))],
        out_specs=pl.BlockSpec((H, 128), lambda g: (g, 0)),
        out_shape=jax.ShapeDtypeStruct((G * H, 128), jnp.float32),
    )(xt)


def _sc_body(idx_hbm, enc_hbm, dec_hbm, out_hbm,
             idx_v0, idx_v1, rows_v0, rows_v1, out_v0, out_v1,
             gsem0, gsem1, osem0, osem1):
    wid = lax.axis_index("s") * NC + lax.axis_index("c")
    t0 = wid * CPW
    lane = lax.iota(jnp.int32, 16)

    idx_bufs = (idx_v0, idx_v1)
    rows_bufs = (rows_v0, rows_v1)
    out_bufs = (out_v0, out_v1)
    gsems = (gsem0, gsem1)
    osems = (osem0, osem1)

    def gather_copies(k):
        idx_v, rows_v, gsem = idx_bufs[k], rows_bufs[k], gsems[k]
        return [
            pltpu.make_async_copy(enc_hbm.at[idx_v.at[pl.ds(0, CB)]],
                                  rows_v.at[pl.ds(0, CB)], gsem),
            pltpu.make_async_copy(dec_hbm.at[idx_v.at[pl.ds(CB, 128)]],
                                  rows_v.at[pl.ds(CB, 128)], gsem),
            pltpu.make_async_copy(dec_hbm.at[idx_v.at[pl.ds(CB + 128, 128)]],
                                  rows_v.at[pl.ds(CB + 128, 128)], gsem),
            pltpu.make_async_copy(dec_hbm.at[idx_v.at[pl.ds(CB + 256, NPK - CB - 256)]],
                                  rows_v.at[pl.ds(CB + 256, NPK - CB - 256)], gsem),
        ]

    def fire(t, k):
        pltpu.sync_copy(idx_hbm.at[pl.ds(t * IDX_STRIDE, IDX_STRIDE)],
                        idx_bufs[k])
        for cp in gather_copies(k):
            cp.start()

    def compute(k):
        idx_v, rows_v, out_v = idx_bufs[k], rows_bufs[k], out_bufs[k]

        def elem_body(e, _):
            offs1 = idx_v[pl.ds(OFF0 + e * 32, 16)]
            offs2 = idx_v[pl.ds(OFF0 + e * 32 + 16, 16)]
            woff = offs1[0] & 64
            w = [rows_v[e, pl.ds(woff + q * 16, 16)] for q in range(NQ)]
            acc_lo = jnp.zeros((16,), jnp.float32)
            acc_hi = jnp.zeros((16,), jnp.float32)
            for j in range(NJ):
                joff = (offs1[1 + j] if j < 15 else offs2[j - 15]) & 64
                row = CB + j * 16 + e
                r0 = rows_v[row, pl.ds(joff, 16)] * w[0]
                r1 = rows_v[row, pl.ds(joff + 16, 16)] * w[1]
                r2 = rows_v[row, pl.ds(joff + 32, 16)] * w[2]
                r3 = rows_v[row, pl.ds(joff + 48, 16)] * w[3]
                s = jnp.sum((r0 + r1) + (r2 + r3))
                if j < 16:
                    acc_lo = jnp.where(lane == j, s, acc_lo)
                else:
                    acc_hi = jnp.where(lane == j - 16, s, acc_hi)
            pos0 = e * 32
            plsc.store_scatter(out_v, [pos0 + lane], acc_lo)
            plsc.store_scatter(out_v, [pos0 + 16 + lane], acc_hi,
                               mask=lane < NJ - 16)
            return ()

        lax.fori_loop(0, CB, elem_body, (), unroll=False)

    def out_copy(t, k):
        return pltpu.make_async_copy(
            out_bufs[k], out_hbm.at[pl.ds(t * OUT_STRIDE, 512)], osems[k])

    def outer(c, _):
        t = t0 + c
        fire(t, 0)
        for cp in gather_copies(0):
            cp.wait()
        compute(0)
        cp = out_copy(t, 0)
        cp.start()
        cp.wait()
        return ()

    lax.fori_loop(0, CPW, outer, (), unroll=False)


@jax.jit
def _run(idx_flat, enc_t, dec_t):
    enc_p = _tc_pack(enc_t)
    dec_p = _tc_pack(dec_t)
    mesh = plsc.VectorSubcoreMesh(
        core_axis_name="c", subcore_axis_name="s",
        num_cores=NC, num_subcores=NS)
    f = pl.kernel(
        _sc_body,
        out_type=jax.ShapeDtypeStruct((T_CHUNKS * OUT_STRIDE,), jnp.float32),
        mesh=mesh,
        scratch_types=[
            pltpu.VMEM((IDX_STRIDE,), jnp.int32),
            pltpu.VMEM((IDX_STRIDE,), jnp.int32),
            pltpu.VMEM((NPK, 128), jnp.float32),
            pltpu.VMEM((NPK, 128), jnp.float32),
            pltpu.VMEM((512,), jnp.float32),
            pltpu.VMEM((512,), jnp.float32),
            pltpu.SemaphoreType.DMA,
            pltpu.SemaphoreType.DMA,
            pltpu.SemaphoreType.DMA,
            pltpu.SemaphoreType.DMA,
        ],
        compiler_params=pltpu.CompilerParams(needs_layout_passes=False),
    )
    return f(idx_flat, enc_p, dec_p)


def kernel(input_tokens, ctx_tokens, neg_tokens, encoder_weight, decoder_weight):
    all_idx = jnp.concatenate(
        [input_tokens.reshape(1, BATCH),
         ctx_tokens.reshape(1, BATCH),
         neg_tokens.T], axis=0).astype(jnp.int32)
    p_all = (all_idx >> 11) * H + (all_idx & (H - 1))
    off_all = ((all_idx >> 10) & 1) * 64
    pblk = (p_all.reshape(NR, T_CHUNKS, CB)
            .transpose(1, 0, 2).reshape(T_CHUNKS, NPK))
    offe = (off_all.reshape(NR, T_CHUNKS, CB)
            .transpose(1, 2, 0))
    offe = jnp.pad(offe, ((0, 0), (0, 0), (0, 32 - NR)))
    offblk = offe.reshape(T_CHUNKS, CB * 32)
    pad = jnp.zeros((T_CHUNKS, IDX_STRIDE - NPK - CB * 32), jnp.int32)
    idx_flat = jnp.concatenate([pblk, offblk, pad], axis=1).reshape(-1)

    out = _run(idx_flat, encoder_weight.T, decoder_weight.T)
    out = out.reshape(T_CHUNKS, 32, 32)[:, :CB, :NJ].reshape(BATCH, NJ)
    pos = out[:, 0].reshape(BATCH, 1, 1)
    neg = out[:, 1:].reshape(BATCH, 1, NEG)
    return (pos, neg)


# speedup vs baseline: 9.3051x; 1.6385x over previous
import math

import jax
import jax.numpy as jnp
from jax import lax
from jax.experimental import pallas as pl
from jax.experimental.pallas import tpu as pltpu
from jax.experimental.pallas import tpu_sc as plsc

VOCAB = 1000000
EMB = 64
BATCH = 16384
NEG = 20
NJ = 1 + NEG
NR = 1 + NJ

W = 8192
H = W // 2
G = math.ceil(VOCAB / W)

NC = 2
NS = 16
NW = NC * NS
CB = 16
T_CHUNKS = BATCH // CB
CPW = T_CHUNKS // NW
IDX_STRIDE = 1024
OUT_STRIDE = 1024
NPK = NR * CB
OFF0 = NPK
NQ = EMB // 16


def _tc_pack_body(x_ref, o_ref):
    a = x_ref[:, :H].T
    b = x_ref[:, H:].T
    o_ref[...] = jnp.concatenate([a, b], axis=1)


def _tc_pack(xt):
    return pl.pallas_call(
        _tc_pack_body,
        grid=(G,),
        in_specs=[pl.BlockSpec((64, W), lambda g: (0, g))],
        out_specs=pl.BlockSpec((H, 128), lambda g: (g, 0)),
        out_shape=jax.ShapeDtypeStruct((G * H, 128), jnp.float32),
    )(xt)


def _sc_body(idx_hbm, enc_hbm, dec_hbm, out_hbm,
             idx_v0, idx_v1, rows_v0, rows_v1, out_v0, out_v1,
             gsem0, gsem1, osem0, osem1, isem0, isem1):
    wid = lax.axis_index("s") * NC + lax.axis_index("c")
    t0 = wid * CPW
    lane = lax.iota(jnp.int32, 16)

    idx_bufs = (idx_v0, idx_v1)
    rows_bufs = (rows_v0, rows_v1)
    out_bufs = (out_v0, out_v1)
    gsems = (gsem0, gsem1)
    osems = (osem0, osem1)
    isems = (isem0, isem1)

    def gather_copies(k):
        idx_v, rows_v, gsem = idx_bufs[k], rows_bufs[k], gsems[k]
        return [
            pltpu.make_async_copy(enc_hbm.at[idx_v.at[pl.ds(0, CB)]],
                                  rows_v.at[pl.ds(0, CB)], gsem),
            pltpu.make_async_copy(dec_hbm.at[idx_v.at[pl.ds(CB, 128)]],
                                  rows_v.at[pl.ds(CB, 128)], gsem),
            pltpu.make_async_copy(dec_hbm.at[idx_v.at[pl.ds(CB + 128, 128)]],
                                  rows_v.at[pl.ds(CB + 128, 128)], gsem),
            pltpu.make_async_copy(dec_hbm.at[idx_v.at[pl.ds(CB + 256, NPK - CB - 256)]],
                                  rows_v.at[pl.ds(CB + 256, NPK - CB - 256)], gsem),
        ]

    def idx_copy(t, k):
        return pltpu.make_async_copy(
            idx_hbm.at[pl.ds(t * IDX_STRIDE, IDX_STRIDE)], idx_bufs[k],
            isems[k])

    def compute(k):
        idx_v, rows_v, out_v = idx_bufs[k], rows_bufs[k], out_bufs[k]

        def elem_body(e, _):
            offs1 = idx_v[pl.ds(OFF0 + e * 32, 16)]
            offs2 = idx_v[pl.ds(OFF0 + e * 32 + 16, 16)]
            woff = offs1[0] & 64
            w = [rows_v[e, pl.ds(woff + q * 16, 16)] for q in range(NQ)]
            acc_lo = jnp.zeros((16,), jnp.float32)
            acc_hi = jnp.zeros((16,), jnp.float32)
            for j in range(NJ):
                joff = (offs1[1 + j] if j < 15 else offs2[j - 15]) & 64
                row = CB + j * 16 + e
                r0 = rows_v[row, pl.ds(joff, 16)] * w[0]
                r1 = rows_v[row, pl.ds(joff + 16, 16)] * w[1]
                r2 = rows_v[row, pl.ds(joff + 32, 16)] * w[2]
                r3 = rows_v[row, pl.ds(joff + 48, 16)] * w[3]
                s = jnp.sum((r0 + r1) + (r2 + r3))
                if j < 16:
                    acc_lo = jnp.where(lane == j, s, acc_lo)
                else:
                    acc_hi = jnp.where(lane == j - 16, s, acc_hi)
            pos0 = e * 32
            plsc.store_scatter(out_v, [pos0 + lane], acc_lo)
            plsc.store_scatter(out_v, [pos0 + 16 + lane], acc_hi,
                               mask=lane < NJ - 16)
            return ()

        lax.fori_loop(0, CB, elem_body, (), unroll=False)

    def out_copy(t, k):
        return pltpu.make_async_copy(
            out_bufs[k], out_hbm.at[pl.ds(t * OUT_STRIDE, 512)], osems[k])

    idx_copy(t0, 0).start()
    idx_copy(t0, 0).wait()
    for cp in gather_copies(0):
        cp.start()
    idx_copy(t0 + 1, 1).start()

    def outer(c2, _):
        for b in range(2):
            c = c2 * 2 + b
            t = t0 + c
            for cp in gather_copies(b):
                cp.wait()

            @pl.when(c + 1 < CPW)
            def _():
                idx_copy(t + 1, 1 - b).wait()
                for cp in gather_copies(1 - b):
                    cp.start()

            @pl.when(c >= 2)
            def _():
                out_copy(t - 2, b).wait()

            compute(b)
            out_copy(t, b).start()

            @pl.when(c + 2 < CPW)
            def _():
                idx_copy(t + 2, b).start()
        return ()

    lax.fori_loop(0, CPW // 2, outer, (), unroll=False)
    out_copy(t0 + CPW - 2, 0).wait()
    out_copy(t0 + CPW - 1, 1).wait()


@jax.jit
def _run(idx_flat, enc_t, dec_t):
    enc_p = _tc_pack(enc_t)
    dec_p = _tc_pack(dec_t)
    mesh = plsc.VectorSubcoreMesh(
        core_axis_name="c", subcore_axis_name="s",
        num_cores=NC, num_subcores=NS)
    f = pl.kernel(
        _sc_body,
        out_type=jax.ShapeDtypeStruct((T_CHUNKS * OUT_STRIDE,), jnp.float32),
        mesh=mesh,
        scratch_types=[
            pltpu.VMEM((IDX_STRIDE,), jnp.int32),
            pltpu.VMEM((IDX_STRIDE,), jnp.int32),
            pltpu.VMEM((NPK, 128), jnp.float32),
            pltpu.VMEM((NPK, 128), jnp.float32),
            pltpu.VMEM((512,), jnp.float32),
            pltpu.VMEM((512,), jnp.float32),
            pltpu.SemaphoreType.DMA,
            pltpu.SemaphoreType.DMA,
            pltpu.SemaphoreType.DMA,
            pltpu.SemaphoreType.DMA,
            pltpu.SemaphoreType.DMA,
            pltpu.SemaphoreType.DMA,
        ],
        compiler_params=pltpu.CompilerParams(needs_layout_passes=False),
    )
    return f(idx_flat, enc_p, dec_p)


def kernel(input_tokens, ctx_tokens, neg_tokens, encoder_weight, decoder_weight):
    all_idx = jnp.concatenate(
        [input_tokens.reshape(1, BATCH),
         ctx_tokens.reshape(1, BATCH),
         neg_tokens.T], axis=0).astype(jnp.int32)
    p_all = (all_idx >> 13) * H + (all_idx & (H - 1))
    off_all = ((all_idx >> 12) & 1) * 64
    pblk = (p_all.reshape(NR, T_CHUNKS, CB)
            .transpose(1, 0, 2).reshape(T_CHUNKS, NPK))
    offe = (off_all.reshape(NR, T_CHUNKS, CB)
            .transpose(1, 2, 0))
    offe = jnp.pad(offe, ((0, 0), (0, 0), (0, 32 - NR)))
    offblk = offe.reshape(T_CHUNKS, CB * 32)
    pad = jnp.zeros((T_CHUNKS, IDX_STRIDE - NPK - CB * 32), jnp.int32)
    idx_flat = jnp.concatenate([pblk, offblk, pad], axis=1).reshape(-1)

    out = _run(idx_flat, encoder_weight.T, decoder_weight.T)
    out = out.reshape(T_CHUNKS, 32, 32)[:, :CB, :NJ].reshape(BATCH, NJ)
    pos = out[:, 0].reshape(BATCH, 1, 1)
    neg = out[:, 1:].reshape(BATCH, 1, NEG)
    return (pos, neg)


# speedup vs baseline: 9.4029x; 1.0105x over previous
import math

import jax
import jax.numpy as jnp
from jax import lax
from jax.experimental import pallas as pl
from jax.experimental.pallas import tpu as pltpu
from jax.experimental.pallas import tpu_sc as plsc

VOCAB = 1000000
EMB = 64
BATCH = 16384
NEG = 20
NJ = 1 + NEG
NR = 1 + NJ

W = 8192
H = W // 2
G = math.ceil(VOCAB / W)

NC = 2
NS = 16
NW = NC * NS
CB = 16
T_CHUNKS = BATCH // CB
CPW = T_CHUNKS // NW
IDX_STRIDE = 768
OUT_STRIDE = 512
NPD = NJ * CB
PW0 = NPD
NPK = NPD + CB
OFFD = NPK
OFFW = OFFD + NPD
NQ = EMB // 16


def _tc_idxprep_body(w_ref, dec_ref, o_ref):
    w = w_ref[...]
    dec = dec_ref[...]
    pdec = (dec >> 13) * H + (dec & (H - 1))
    pw = (w >> 13) * H + (w & (H - 1))
    odec = ((dec >> 12) & 1) << 6
    ow = ((w >> 12) & 1) << 6
    z = jnp.zeros((128, IDX_STRIDE - OFFW - CB), jnp.int32)
    o_ref[...] = jnp.concatenate([pdec, pw, odec, ow, z], axis=1)


def _tc_idxprep(word2d, dec2d):
    return pl.pallas_call(
        _tc_idxprep_body,
        grid=(8,),
        in_specs=[
            pl.BlockSpec((128, CB), lambda g: (g, 0)),
            pl.BlockSpec((128, NPD), lambda g: (g, 0)),
        ],
        out_specs=pl.BlockSpec((128, IDX_STRIDE), lambda g: (g, 0)),
        out_shape=jax.ShapeDtypeStruct((T_CHUNKS, IDX_STRIDE), jnp.int32),
    )(word2d, dec2d)


def _tc_pack_body(x_ref, o_ref):
    a = x_ref[:, :H].T
    b = x_ref[:, H:].T
    o_ref[...] = jnp.concatenate([a, b], axis=1)


def _tc_pack(xt):
    return pl.pallas_call(
        _tc_pack_body,
        grid=(G,),
        in_specs=[pl.BlockSpec((64, W), lambda g: (0, g))],
        out_specs=pl.BlockSpec((H, 128), lambda g: (g, 0)),
        out_shape=jax.ShapeDtypeStruct((G * H, 128), jnp.float32),
    )(xt)


def _sc_body(idx_hbm, enc_hbm, dec_hbm, out_hbm,
             idx_v0, idx_v1, rows_v0, rows_v1, out_v0, out_v1,
             gsem0, gsem1, osem0, osem1, isem0, isem1):
    wid = lax.axis_index("s") * NC + lax.axis_index("c")
    t0 = wid * CPW
    lane = lax.iota(jnp.int32, 16)

    idx_bufs = (idx_v0, idx_v1)
    rows_bufs = (rows_v0, rows_v1)
    out_bufs = (out_v0, out_v1)
    gsems = (gsem0, gsem1)
    osems = (osem0, osem1)
    isems = (isem0, isem1)

    def gather_copies(k):
        idx_v, rows_v, gsem = idx_bufs[k], rows_bufs[k], gsems[k]
        return [
            pltpu.make_async_copy(dec_hbm.at[idx_v.at[pl.ds(0, 128)]],
                                  rows_v.at[pl.ds(0, 128)], gsem),
            pltpu.make_async_copy(dec_hbm.at[idx_v.at[pl.ds(128, 128)]],
                                  rows_v.at[pl.ds(128, 128)], gsem),
            pltpu.make_async_copy(dec_hbm.at[idx_v.at[pl.ds(256, NPD - 256)]],
                                  rows_v.at[pl.ds(256, NPD - 256)], gsem),
            pltpu.make_async_copy(enc_hbm.at[idx_v.at[pl.ds(PW0, CB)]],
                                  rows_v.at[pl.ds(PW0, CB)], gsem),
        ]

    def idx_copy(t, k):
        return pltpu.make_async_copy(
            idx_hbm.at[pl.ds(t * IDX_STRIDE, IDX_STRIDE)], idx_bufs[k],
            isems[k])

    def compute(k):
        idx_v, rows_v, out_v = idx_bufs[k], rows_bufs[k], out_bufs[k]

        def elem_body(e, _):
            offs1 = idx_v[pl.ds(OFFD + e * NJ, 16)]
            offs2 = idx_v[pl.ds(OFFD + e * NJ + 16, 16)]
            woff = idx_v[pl.ds(OFFW + e, 16)][0] & 64
            wrow = PW0 + e
            w = [rows_v[wrow, pl.ds(woff + q * 16, 16)] for q in range(NQ)]
            acc_lo = jnp.zeros((16,), jnp.float32)
            acc_hi = jnp.zeros((16,), jnp.float32)
            for j in range(NJ):
                joff = (offs1[j] if j < 16 else offs2[j - 16]) & 64
                row = e * NJ + j
                r0 = rows_v[row, pl.ds(joff, 16)] * w[0]
                r1 = rows_v[row, pl.ds(joff + 16, 16)] * w[1]
                r2 = rows_v[row, pl.ds(joff + 32, 16)] * w[2]
                r3 = rows_v[row, pl.ds(joff + 48, 16)] * w[3]
                s = jnp.sum((r0 + r1) + (r2 + r3))
                if j < 16:
                    acc_lo = jnp.where(lane == j, s, acc_lo)
                else:
                    acc_hi = jnp.where(lane == j - 16, s, acc_hi)
            pos0 = e * 32
            plsc.store_scatter(out_v, [pos0 + lane], acc_lo)
            plsc.store_scatter(out_v, [pos0 + 16 + lane], acc_hi,
                               mask=lane < NJ - 16)
            return ()

        lax.fori_loop(0, CB, elem_body, (), unroll=False)

    def out_copy(t, k):
        return pltpu.make_async_copy(
            out_bufs[k], out_hbm.at[pl.ds(t * OUT_STRIDE, OUT_STRIDE)],
            osems[k])

    idx_copy(t0, 0).start()
    idx_copy(t0, 0).wait()
    for cp in gather_copies(0):
        cp.start()
    idx_copy(t0 + 1, 1).start()

    def outer(c2, _):
        for b in range(2):
            c = c2 * 2 + b
            t = t0 + c
            for cp in gather_copies(b):
                cp.wait()

            @pl.when(c + 1 < CPW)
            def _():
                idx_copy(t + 1, 1 - b).wait()
                for cp in gather_copies(1 - b):
                    cp.start()

            @pl.when(c >= 2)
            def _():
                out_copy(t - 2, b).wait()

            compute(b)
            out_copy(t, b).start()

            @pl.when(c + 2 < CPW)
            def _():
                idx_copy(t + 2, b).start()
        return ()

    lax.fori_loop(0, CPW // 2, outer, (), unroll=False)
    out_copy(t0 + CPW - 2, 0).wait()
    out_copy(t0 + CPW - 1, 1).wait()


@jax.jit
def _run(word2d, dec2d, enc_t, dec_t):
    idx_flat = _tc_idxprep(word2d, dec2d).reshape(-1)
    enc_p = _tc_pack(enc_t)
    dec_p = _tc_pack(dec_t)
    mesh = plsc.VectorSubcoreMesh(
        core_axis_name="c", subcore_axis_name="s",
        num_cores=NC, num_subcores=NS)
    f = pl.kernel(
        _sc_body,
        out_type=jax.ShapeDtypeStruct((T_CHUNKS * OUT_STRIDE,), jnp.float32),
        mesh=mesh,
        scratch_types=[
            pltpu.VMEM((IDX_STRIDE,), jnp.int32),
            pltpu.VMEM((IDX_STRIDE,), jnp.int32),
            pltpu.VMEM((NPK, 128), jnp.float32),
            pltpu.VMEM((NPK, 128), jnp.float32),
            pltpu.VMEM((OUT_STRIDE,), jnp.float32),
            pltpu.VMEM((OUT_STRIDE,), jnp.float32),
            pltpu.SemaphoreType.DMA,
            pltpu.SemaphoreType.DMA,
            pltpu.SemaphoreType.DMA,
            pltpu.SemaphoreType.DMA,
            pltpu.SemaphoreType.DMA,
            pltpu.SemaphoreType.DMA,
        ],
        compiler_params=pltpu.CompilerParams(needs_layout_passes=False),
    )
    return f(idx_flat, enc_p, dec_p)


def kernel(input_tokens, ctx_tokens, neg_tokens, encoder_weight, decoder_weight):
    word2d = input_tokens.astype(jnp.int32).reshape(T_CHUNKS, CB)
    dec2d = jnp.concatenate(
        [ctx_tokens.astype(jnp.int32), neg_tokens.astype(jnp.int32)],
        axis=1).reshape(T_CHUNKS, NPD)
    out = _run(word2d, dec2d, encoder_weight.T, decoder_weight.T)
    out = out.reshape(T_CHUNKS, CB, 32)[:, :, :NJ].reshape(BATCH, NJ)
    pos = out[:, 0].reshape(BATCH, 1, 1)
    neg = out[:, 1:].reshape(BATCH, 1, NEG)
    return (pos, neg)
